# Initial kernel scaffold; baseline (speedup 1.0000x reference)
import jax
import jax.numpy as jnp
from jax.experimental import pallas as pl


def kernel(x, table):
    raise NotImplementedError("write your pallas kernel here")


# speedup vs baseline: 1.8365x; 1.8365x over previous
import jax
import jax.numpy as jnp
from jax import lax
from jax.experimental import pallas as pl
from jax.experimental.pallas import tpu as pltpu
from jax.experimental.pallas import tpu_sc as plsc

VOCAB = 1000000
EMBED_DIM = 64
BATCH = 16384
HIST = 50

_NC = 2
_NS = 16
_NW = _NC * _NS

_N_ROWS = BATCH * HIST
_PER_W = _N_ROWS // _NW
_GRP = 128
_NGRP = _PER_W // _GRP


def _sc_gather(idx, table):
    mesh = plsc.VectorSubcoreMesh(core_axis_name="c", subcore_axis_name="s")

    @pl.kernel(
        out_type=jax.ShapeDtypeStruct((_N_ROWS, EMBED_DIM), jnp.float32),
        mesh=mesh,
        compiler_params=pltpu.CompilerParams(use_tc_tiling_on_sc=False),
        scratch_types=[
            pltpu.VMEM((_NGRP, _GRP), jnp.int32),
            pltpu.VMEM((_GRP, EMBED_DIM), jnp.float32),
            pltpu.VMEM((_GRP, EMBED_DIM), jnp.float32),
            pltpu.SemaphoreType.DMA,
            pltpu.SemaphoreType.DMA,
        ],
    )
    def k(idx_hbm, table_hbm, out_hbm, idx_v, rows_a, rows_b, sem_a, sem_b):
        wid = lax.axis_index("s") * _NC + lax.axis_index("c")
        base = wid * _PER_W
        pltpu.sync_copy(idx_hbm.at[wid], idx_v)

        def wait_a():
            pltpu.make_async_copy(
                table_hbm.at[pl.ds(0, _GRP)], rows_a, sem_a).wait()

        def wait_b():
            pltpu.make_async_copy(
                table_hbm.at[pl.ds(0, _GRP)], rows_b, sem_b).wait()

        pltpu.async_copy(table_hbm.at[idx_v.at[0]], rows_a, sem_a)

        def step(g, carry):
            j = 2 * g
            pltpu.async_copy(table_hbm.at[idx_v.at[j + 1]], rows_b, sem_b)
            wait_a()
            pltpu.sync_copy(rows_a, out_hbm.at[pl.ds(base + j * _GRP, _GRP)])
            jn = jnp.minimum(j + 2, _NGRP - 2)
            pltpu.async_copy(table_hbm.at[idx_v.at[jn]], rows_a, sem_a)
            wait_b()
            pltpu.sync_copy(
                rows_b, out_hbm.at[pl.ds(base + (j + 1) * _GRP, _GRP)])
            return carry

        lax.fori_loop(0, _NGRP // 2, step, 0)
        wait_a()

    return k(idx, table)


def kernel(x, table):
    idx = x.reshape(_NW, _NGRP, _GRP).astype(jnp.int32)
    out = _sc_gather(idx, table)
    return out.reshape(BATCH, HIST, EMBED_DIM)


# speedup vs baseline: 9.1456x; 4.9800x over previous
import jax
import jax.numpy as jnp
from jax import lax
from jax.experimental import pallas as pl
from jax.experimental.pallas import tpu as pltpu
from jax.experimental.pallas import tpu_sc as plsc

VOCAB = 1000000
EMBED_DIM = 64
BATCH = 16384
HIST = 50

_NC = 2
_NS = 16
_NW = _NC * _NS

_N_ROWS = BATCH * HIST
_PER_W = _N_ROWS // _NW
_GRP = 512
_NGRP = _PER_W // _GRP


def _sc_gather(idx, table):
    mesh = plsc.VectorSubcoreMesh(core_axis_name="c", subcore_axis_name="s")

    @pl.kernel(
        out_type=jax.ShapeDtypeStruct((_N_ROWS, EMBED_DIM), jnp.float32),
        mesh=mesh,
        compiler_params=pltpu.CompilerParams(use_tc_tiling_on_sc=False),
        scratch_types=[
            pltpu.VMEM((_NGRP, _GRP), jnp.int32),
            pltpu.VMEM((_GRP, EMBED_DIM), jnp.float32),
            pltpu.VMEM((_GRP, EMBED_DIM), jnp.float32),
            pltpu.SemaphoreType.DMA,
            pltpu.SemaphoreType.DMA,
        ],
    )
    def k(idx_hbm, table_hbm, out_hbm, idx_v, rows_a, rows_b, sem_a, sem_b):
        wid = lax.axis_index("s") * _NC + lax.axis_index("c")
        base = wid * _PER_W
        pltpu.sync_copy(idx_hbm.at[wid], idx_v)

        def wait_a():
            pltpu.make_async_copy(
                table_hbm.at[pl.ds(0, _GRP)], rows_a, sem_a).wait()

        def wait_b():
            pltpu.make_async_copy(
                table_hbm.at[pl.ds(0, _GRP)], rows_b, sem_b).wait()

        pltpu.async_copy(table_hbm.at[idx_v.at[0]], rows_a, sem_a)

        def step(g, carry):
            j = 2 * g
            pltpu.async_copy(table_hbm.at[idx_v.at[j + 1]], rows_b, sem_b)
            wait_a()
            pltpu.sync_copy(rows_a, out_hbm.at[pl.ds(base + j * _GRP, _GRP)])
            jn = jnp.minimum(j + 2, _NGRP - 2)
            pltpu.async_copy(table_hbm.at[idx_v.at[jn]], rows_a, sem_a)
            wait_b()
            pltpu.sync_copy(
                rows_b, out_hbm.at[pl.ds(base + (j + 1) * _GRP, _GRP)])
            return carry

        lax.fori_loop(0, _NGRP // 2, step, 0)
        wait_a()

    return k(idx, table)


def kernel(x, table):
    idx = x.reshape(_NW, _NGRP, _GRP).astype(jnp.int32)
    out = _sc_gather(idx, table)
    return out.reshape(BATCH, HIST, EMBED_DIM)
